# baseline (speedup 1.0000x reference)
import functools

import jax
import jax.numpy as jnp
from jax import lax
from jax.experimental import pallas as pl
from jax.experimental.pallas import tpu as pltpu
from jax.experimental.pallas import tpu_sc as plsc

MAX_CONTEXT = 8192
EMBEDDING_DIM = 128
NUM_CORES = 2
NUM_SUBCORES = 16
NUM_WORKERS = NUM_CORES * NUM_SUBCORES
RPW = MAX_CONTEXT // NUM_WORKERS
GROUPS_PER_BATCH = RPW // 16


def _make_body(B, S):
    NG = B * GROUPS_PER_BATCH

    HALF = RPW // 2

    def body(in_hbm, emb_hbm, out_hbm, emb_v, in_v, idx2d, tmp_v, zeros_v,
             sem_in, sem_e0, sem_e1, sem_o, sem_f):
        wid = lax.axis_index("s") * NUM_CORES + lax.axis_index("c")
        s0 = wid * RPW

        h_e0 = pltpu.async_copy(
            emb_hbm.at[pl.ds(s0, HALF), :],
            emb_v.at[pl.ds(0, HALF), :], sem_e0)
        h_e1 = pltpu.async_copy(
            emb_hbm.at[pl.ds(s0 + HALF, HALF), :],
            emb_v.at[pl.ds(HALF, HALF), :], sem_e1)
        h_in = [
            pltpu.async_copy(
                in_hbm.at[b, pl.ds(s0, RPW)],
                in_v.at[pl.ds(b * RPW, RPW)],
                sem_in,
            )
            for b in range(B)
        ]
        for h in h_in:
            h.wait()

        iota16 = lax.iota(jnp.int32, 16)

        def scan(g, dcnt):
            v = in_v[pl.ds(g * 16, 16)]
            z = v == 0
            cnt = plsc.all_reduce_population_count(z)[0]

            @pl.when(cnt > 0)
            def _():
                b = g // GROUPS_PER_BATCH
                rbase = (g % GROUPS_PER_BATCH) * 16
                rows = b * S + s0 + rbase + iota16
                plsc.store_compressed(tmp_v.at[pl.ds(0, 16)], rows, mask=z)
                first = tmp_v[pl.ds(0, 16)][0]
                idx2d[dcnt, :] = jnp.where(z, rows, first)

            return dcnt + (cnt > 0).astype(jnp.int32)

        dcnt = lax.fori_loop(0, NG, scan, 0)

        h_e0.wait()
        h_out = [
            pltpu.async_copy(
                emb_v.at[pl.ds(0, HALF), :],
                out_hbm.at[pl.ds(b * S + s0, HALF), :], sem_o)
            for b in range(B)
        ]
        h_e1.wait()
        h_out += [
            pltpu.async_copy(
                emb_v.at[pl.ds(HALF, HALF), :],
                out_hbm.at[pl.ds(b * S + s0 + HALF, HALF), :], sem_o)
            for b in range(B)
        ]
        for h in h_out:
            h.wait()

        @pl.when(dcnt > 0)
        def _():
            def zinit(i, c):
                for j in range(EMBEDDING_DIM // 16):
                    zeros_v[i, pl.ds(j * 16, 16)] = jnp.zeros(
                        (16,), jnp.float32)
                return c

            lax.fori_loop(0, 16, zinit, 0)

        def fix(j, c):
            pltpu.async_copy(
                zeros_v, out_hbm.at[idx2d.at[j]], sem_f).wait()
            return c

        lax.fori_loop(0, dcnt, fix, 0)

    return body


@jax.jit
def _run(inputs, embedding_matrix):
    B, S = inputs.shape
    mesh = plsc.VectorSubcoreMesh(core_axis_name="c", subcore_axis_name="s")
    fn = functools.partial(
        pl.kernel,
        mesh=mesh,
        compiler_params=pltpu.CompilerParams(needs_layout_passes=False),
        out_type=jax.ShapeDtypeStruct((B * S, EMBEDDING_DIM), jnp.float32),
        scratch_types=[
            pltpu.VMEM((RPW, EMBEDDING_DIM), jnp.float32),
            pltpu.VMEM((B * RPW,), jnp.int32),
            pltpu.VMEM((B * GROUPS_PER_BATCH, 16), jnp.int32),
            pltpu.VMEM((16,), jnp.int32),
            pltpu.VMEM((16, EMBEDDING_DIM), jnp.float32),
            pltpu.SemaphoreType.DMA,
            pltpu.SemaphoreType.DMA,
            pltpu.SemaphoreType.DMA,
            pltpu.SemaphoreType.DMA,
            pltpu.SemaphoreType.DMA,
        ],
    )(_make_body(B, S))
    out = fn(inputs, embedding_matrix)
    return out.reshape(B, S, EMBEDDING_DIM)


def kernel(inputs, embedding_matrix):
    if inputs.shape[1] > MAX_CONTEXT:
        inputs = inputs[:, -MAX_CONTEXT:]
    return _run(inputs.astype(jnp.int32), embedding_matrix)


# speedup vs baseline: 1.0094x; 1.0094x over previous
import functools

import jax
import jax.numpy as jnp
from jax import lax
from jax.experimental import pallas as pl
from jax.experimental.pallas import tpu as pltpu
from jax.experimental.pallas import tpu_sc as plsc

MAX_CONTEXT = 8192
EMBEDDING_DIM = 128
NUM_CORES = 2
NUM_SUBCORES = 16
NUM_WORKERS = NUM_CORES * NUM_SUBCORES
RPW = MAX_CONTEXT // NUM_WORKERS
GROUPS_PER_BATCH = RPW // 16


def _make_body(B, S):
    NG = B * GROUPS_PER_BATCH

    def body(in_hbm, emb_hbm, out_hbm, emb_v, in_v, idx2d, tmp_v, zeros_v,
             sem_in, sem_emb, sem_o, sem_f):
        wid = lax.axis_index("s") * NUM_CORES + lax.axis_index("c")
        s0 = wid * RPW

        h_emb = pltpu.async_copy(emb_hbm.at[pl.ds(s0, RPW), :], emb_v,
                                 sem_emb)
        h_in = [
            pltpu.async_copy(
                in_hbm.at[b, pl.ds(s0, RPW)],
                in_v.at[pl.ds(b * RPW, RPW)],
                sem_in,
            )
            for b in range(B)
        ]
        for h in h_in:
            h.wait()

        iota16 = lax.iota(jnp.int32, 16)

        def scan(g, dcnt):
            v = in_v[pl.ds(g * 16, 16)]
            z = v == 0
            cnt = plsc.all_reduce_population_count(z)[0]

            @pl.when(cnt > 0)
            def _():
                b = g // GROUPS_PER_BATCH
                rbase = (g % GROUPS_PER_BATCH) * 16
                rows = b * S + s0 + rbase + iota16
                plsc.store_compressed(tmp_v.at[pl.ds(0, 16)], rows, mask=z)
                first = tmp_v[pl.ds(0, 16)][0]
                idx2d[dcnt, :] = jnp.where(z, rows, first)

            return dcnt + (cnt > 0).astype(jnp.int32)

        dcnt = lax.fori_loop(0, NG, scan, 0)

        h_emb.wait()
        h_out = [
            pltpu.async_copy(
                emb_v, out_hbm.at[pl.ds(b * S + s0, RPW), :], sem_o)
            for b in range(B)
        ]
        for h in h_out:
            h.wait()

        @pl.when(dcnt > 0)
        def _():
            def zinit(i, c):
                for j in range(EMBEDDING_DIM // 16):
                    zeros_v[i, pl.ds(j * 16, 16)] = jnp.zeros(
                        (16,), jnp.float32)
                return c

            lax.fori_loop(0, 16, zinit, 0)

        def fix(j, c):
            pltpu.async_copy(
                zeros_v, out_hbm.at[idx2d.at[j]], sem_f).wait()
            return c

        lax.fori_loop(0, dcnt, fix, 0)

    return body


@jax.jit
def _run(inputs, embedding_matrix):
    B, S = inputs.shape
    mesh = plsc.VectorSubcoreMesh(core_axis_name="c", subcore_axis_name="s")
    fn = functools.partial(
        pl.kernel,
        mesh=mesh,
        compiler_params=pltpu.CompilerParams(needs_layout_passes=False),
        out_type=jax.ShapeDtypeStruct((B * S, EMBEDDING_DIM), jnp.float32),
        scratch_types=[
            pltpu.VMEM((RPW, EMBEDDING_DIM), jnp.float32),
            pltpu.VMEM((B * RPW,), jnp.int32),
            pltpu.VMEM((B * GROUPS_PER_BATCH, 16), jnp.int32),
            pltpu.VMEM((16,), jnp.int32),
            pltpu.VMEM((16, EMBEDDING_DIM), jnp.float32),
            pltpu.SemaphoreType.DMA,
            pltpu.SemaphoreType.DMA,
            pltpu.SemaphoreType.DMA,
            pltpu.SemaphoreType.DMA,
        ],
    )(_make_body(B, S))
    out = fn(inputs, embedding_matrix)
    return out.reshape(B, S, EMBEDDING_DIM)


def kernel(inputs, embedding_matrix):
    if inputs.shape[1] > MAX_CONTEXT:
        inputs = inputs[:, -MAX_CONTEXT:]
    return _run(inputs.astype(jnp.int32), embedding_matrix)
